# baseline (speedup 1.0000x reference)
import functools

import jax
import jax.numpy as jnp
from jax import lax
from jax.experimental import pallas as pl
from jax.experimental.pallas import tpu as pltpu
from jax.experimental.pallas import tpu_sc as plsc

_NC = 2
_NS = 16
_NW = _NC * _NS
_K = 128
_EPS = 1e-5
_SPLIT0 = 0.25


def _deg_parts(dst2d, zeros1d, ones128, npad, nrows):
    nch = nrows // _NW
    rpt = npad // _NS
    mesh = plsc.VectorSubcoreMesh(core_axis_name="c", subcore_axis_name="s")

    @functools.partial(
        pl.kernel,
        out_type=jax.ShapeDtypeStruct((_NC * npad,), jnp.float32),
        mesh=mesh,
        scratch_types=[
            pltpu.VMEM((nch, _K), jnp.int32),
            pltpu.VMEM((_K,), jnp.float32),
            pltpu.VMEM_SHARED((npad,), jnp.float32),
        ],
    )
    def deg_kernel(dst_hbm, z_hbm, ones_hbm, out_hbm, didx_v, ones_v, acc):
        c = lax.axis_index("c")
        s = lax.axis_index("s")
        wid = c * _NS + s
        r0 = s * rpt
        pltpu.sync_copy(z_hbm.at[pl.ds(r0, rpt)], acc.at[pl.ds(r0, rpt)])
        pltpu.sync_copy(ones_hbm, ones_v)
        pltpu.sync_copy(dst_hbm.at[pl.ds(wid * nch, nch)], didx_v)
        plsc.subcore_barrier()

        @pl.loop(0, nch)
        def _chunk(j):
            pltpu.sync_copy(ones_v, acc.at[didx_v.at[j]], add=True)

        plsc.subcore_barrier()
        pltpu.sync_copy(acc.at[pl.ds(r0, rpt)],
                        out_hbm.at[pl.ds(c * npad + r0, rpt)])

    return deg_kernel(dst2d, zeros1d, ones128)


def _scatter_parts(hs_ext, src2d, dst2d, npad, dout, nch0, nch1):
    nchmax = max(nch0, nch1)
    rpt = npad // _NS
    mesh = plsc.VectorSubcoreMesh(core_axis_name="c", subcore_axis_name="s")

    @functools.partial(
        pl.kernel,
        out_type=jax.ShapeDtypeStruct((_NC, npad, dout), jnp.float32),
        mesh=mesh,
        scratch_types=[
            pltpu.VMEM((nchmax, _K), jnp.int32),
            pltpu.VMEM((_K,), jnp.int32),
            pltpu.VMEM((_K,), jnp.int32),
            pltpu.VMEM((_K, dout), jnp.float32),
            pltpu.VMEM((_K, dout), jnp.float32),
            pltpu.VMEM_SHARED((npad, dout), jnp.float32),
            pltpu.SemaphoreType.DMA,
            pltpu.SemaphoreType.DMA,
            pltpu.SemaphoreType.DMA,
            pltpu.SemaphoreType.DMA,
        ],
    )
    def scat_kernel(hs_hbm, src_hbm, dst_hbm, out_hbm,
                    sidx_v, didx0, didx1, rows0, rows1, acc,
                    gsem0, gsem1, dsem0, dsem1):
        c = lax.axis_index("c")
        s = lax.axis_index("s")
        r0 = s * rpt
        row0 = jnp.where(c == 0, s * nch0, _NS * nch0 + s * nch1)
        ncl = jnp.where(c == 0, nch0, nch1)
        bufs = ((rows0, gsem0, didx0, dsem0), (rows1, gsem1, didx1, dsem1))
        pltpu.sync_copy(hs_hbm.at[pl.ds(r0, rpt)], acc.at[pl.ds(r0, rpt)])
        pltpu.sync_copy(src_hbm.at[pl.ds(row0, nchmax)], sidx_v)
        plsc.subcore_barrier()

        @pl.when(ncl >= 2)
        def _prologue():
            for b, (rows_b, gsem_b, didx_b, dsem_b) in enumerate(bufs):
                pltpu.async_copy(dst_hbm.at[row0 + b], didx_b, dsem_b)
                pltpu.async_copy(hs_hbm.at[sidx_v.at[b]], rows_b, gsem_b)

        @pl.loop(0, ncl - 2, step=2)
        def _chunk(j):
            for b, (rows_b, gsem_b, didx_b, dsem_b) in enumerate(bufs):
                cc = j + b
                pltpu.make_async_copy(dst_hbm.at[row0 + cc],
                                      didx_b, dsem_b).wait()
                pltpu.make_async_copy(hs_hbm.at[sidx_v.at[cc]],
                                      rows_b, gsem_b).wait()
                pltpu.sync_copy(rows_b, acc.at[didx_b], add=True)
                pltpu.async_copy(dst_hbm.at[row0 + cc + 2], didx_b, dsem_b)
                pltpu.async_copy(hs_hbm.at[sidx_v.at[cc + 2]], rows_b, gsem_b)

        @pl.when(ncl >= 2)
        def _epilogue():
            for b, (rows_b, gsem_b, didx_b, dsem_b) in enumerate(bufs):
                cc = ncl - 2 + b
                pltpu.make_async_copy(dst_hbm.at[row0 + cc],
                                      didx_b, dsem_b).wait()
                pltpu.make_async_copy(hs_hbm.at[sidx_v.at[cc]],
                                      rows_b, gsem_b).wait()
                pltpu.sync_copy(rows_b, acc.at[didx_b], add=True)

        plsc.subcore_barrier()
        pltpu.sync_copy(acc.at[pl.ds(r0, rpt)], out_hbm.at[c, pl.ds(r0, rpt)])

    return scat_kernel(hs_ext, src2d, dst2d)


def _to_col(m, blk):
    nr = blk // 128
    rows = lax.broadcasted_iota(jnp.int32, (blk, nr), 0) // 128
    sel = (rows == lax.broadcasted_iota(jnp.int32, (blk, nr), 1))
    c = jnp.dot(sel.astype(jnp.float32), m,
                preferred_element_type=jnp.float32)
    lane = lax.broadcasted_iota(jnp.int32, (blk, 128), 1)
    vmod = lax.broadcasted_iota(jnp.int32, (blk, 128), 0) % 128
    return jnp.sum(jnp.where(lane == vmod, c, 0.0), axis=1, keepdims=True)


def _dinv_block(d0_ref, d1_ref, blk):
    d = _to_col(d0_ref[...] + d1_ref[...] + 1.0, blk)
    return lax.rsqrt(jnp.maximum(d, 1.0))


def _linear(x_ext, W, deg0, deg1, npad, din, dout, blk):
    grid = (npad // blk,)
    dblk = blk // 128

    def body(x_ref, w_ref, d0_ref, d1_ref, o_ref):
        dinv = _dinv_block(d0_ref, d1_ref, blk)
        h = jnp.dot(x_ref[...], w_ref[...], preferred_element_type=jnp.float32)
        o_ref[...] = h * dinv

    return pl.pallas_call(
        body,
        grid=grid,
        in_specs=[
            pl.BlockSpec((blk, din), lambda i: (i, 0)),
            pl.BlockSpec((din, dout), lambda i: (0, 0)),
            pl.BlockSpec((dblk, 128), lambda i: (i, 0)),
            pl.BlockSpec((dblk, 128), lambda i: (i, 0)),
        ],
        out_specs=pl.BlockSpec((blk, dout), lambda i: (i, 0)),
        out_shape=jax.ShapeDtypeStruct((npad, dout), jnp.float32),
    )(x_ext, W, deg0, deg1)


def _agg_block(p0_ref, p1_ref, hs_ref, d0_ref, d1_ref, b_ref, blk):
    dinv = _dinv_block(d0_ref, d1_ref, blk)
    return (p0_ref[...] + p1_ref[...] - hs_ref[...]) * dinv + b_ref[...]


def _stats(p0, p1, hs_ext, deg0, deg1, b2, npad, n, dout, blk):
    grid = (npad // blk,)

    def body(p0_ref, p1_ref, hs_ref, d0_ref, d1_ref, b_ref, o_ref):
        i = pl.program_id(0)
        agg = _agg_block(p0_ref, p1_ref, hs_ref, d0_ref, d1_ref, b_ref, blk)
        rid = lax.broadcasted_iota(jnp.int32, (blk, 1), 0) + i * blk
        agg = jnp.where(rid < n, agg, 0.0)

        @pl.when(i == 0)
        def _():
            o_ref[...] = jnp.zeros_like(o_ref)

        o_ref[0:1, :] += jnp.sum(agg, axis=0, keepdims=True)
        o_ref[1:2, :] += jnp.sum(agg * agg, axis=0, keepdims=True)

    dblk = blk // 128
    return pl.pallas_call(
        body,
        grid=grid,
        in_specs=[
            pl.BlockSpec((blk, dout), lambda i: (i, 0)),
            pl.BlockSpec((blk, dout), lambda i: (i, 0)),
            pl.BlockSpec((blk, dout), lambda i: (i, 0)),
            pl.BlockSpec((dblk, 128), lambda i: (i, 0)),
            pl.BlockSpec((dblk, 128), lambda i: (i, 0)),
            pl.BlockSpec((1, dout), lambda i: (0, 0)),
        ],
        out_specs=pl.BlockSpec((8, dout), lambda i: (0, 0)),
        out_shape=jax.ShapeDtypeStruct((8, dout), jnp.float32),
    )(p0, p1, hs_ext, deg0, deg1, b2)


def _batchnorm(p0, p1, hs_ext, deg0, deg1, b2, g2, bt2, stats, npad, n, dout,
               blk):
    grid = (npad // blk,)
    inv_n = 1.0 / n

    def body(p0_ref, p1_ref, hs_ref, d0_ref, d1_ref, b_ref, g_ref, bt_ref,
             st_ref, o_ref):
        agg = _agg_block(p0_ref, p1_ref, hs_ref, d0_ref, d1_ref, b_ref, blk)
        mean = st_ref[0:1, :] * inv_n
        var = st_ref[1:2, :] * inv_n - mean * mean
        y = (agg - mean) * lax.rsqrt(var + _EPS)
        y = g_ref[...] * y + bt_ref[...]
        o_ref[...] = jnp.maximum(y, 0.0)

    dblk = blk // 128
    return pl.pallas_call(
        body,
        grid=grid,
        in_specs=[
            pl.BlockSpec((blk, dout), lambda i: (i, 0)),
            pl.BlockSpec((blk, dout), lambda i: (i, 0)),
            pl.BlockSpec((blk, dout), lambda i: (i, 0)),
            pl.BlockSpec((dblk, 128), lambda i: (i, 0)),
            pl.BlockSpec((dblk, 128), lambda i: (i, 0)),
            pl.BlockSpec((1, dout), lambda i: (0, 0)),
            pl.BlockSpec((1, dout), lambda i: (0, 0)),
            pl.BlockSpec((1, dout), lambda i: (0, 0)),
            pl.BlockSpec((8, dout), lambda i: (0, 0)),
        ],
        out_specs=pl.BlockSpec((blk, dout), lambda i: (i, 0)),
        out_shape=jax.ShapeDtypeStruct((npad, dout), jnp.float32),
    )(p0, p1, hs_ext, deg0, deg1, b2, g2, bt2, stats)


def kernel(x, edge_index, W, b, gamma, beta):
    n, din = x.shape
    dout = W.shape[1]
    e = edge_index.shape[1]

    blk = 1024
    npad = -(-(n + 1) // blk) * blk
    grp = _NW * _K * 2
    epad = -(-e // grp) * grp
    nrows = epad // _K
    tp = nrows // _NS
    nch0 = min(max(8 * int(round(_SPLIT0 * tp / 8)), 0), tp)
    nch1 = tp - nch0
    nchmax = max(nch0, nch1)

    pad = jnp.full((epad - e + nchmax * _K,), n, dtype=jnp.int32)
    src2d = jnp.concatenate([edge_index[0], pad]).reshape(-1, _K)
    dst2d = jnp.concatenate([edge_index[1], pad]).reshape(-1, _K)
    x_ext = jnp.zeros((npad, din), jnp.float32).at[:n].set(x)
    zeros1d = jnp.zeros((npad,), jnp.float32)
    ones128 = jnp.ones((_K,), jnp.float32)

    degflat = _deg_parts(dst2d, zeros1d, ones128, npad, nrows)
    deg0 = degflat[:npad].reshape(npad // 128, 128)
    deg1 = degflat[npad:].reshape(npad // 128, 128)

    hs_ext = _linear(x_ext, W, deg0, deg1, npad, din, dout, blk)

    parts = _scatter_parts(hs_ext, src2d, dst2d, npad, dout, nch0, nch1)
    p0, p1 = parts[0], parts[1]

    b2 = b.reshape(1, dout)
    g2 = gamma.reshape(1, dout)
    bt2 = beta.reshape(1, dout)
    stats = _stats(p0, p1, hs_ext, deg0, deg1, b2, npad, n, dout, blk)
    y = _batchnorm(p0, p1, hs_ext, deg0, deg1, b2, g2, bt2, stats, npad, n,
                   dout, blk)
    return y[:n]


# speedup vs baseline: 1.0789x; 1.0789x over previous
import functools

import jax
import jax.numpy as jnp
from jax import lax
from jax.experimental import pallas as pl
from jax.experimental.pallas import tpu as pltpu
from jax.experimental.pallas import tpu_sc as plsc

_NC = 2
_NS = 16
_NW = _NC * _NS
_K = 128
_EPS = 1e-5
_SPLIT0 = 0.75


def _deg_parts(dst2d, zeros1d, ones128, npad, nrows):
    nch = nrows // _NW
    rpt = npad // _NS
    mesh = plsc.VectorSubcoreMesh(core_axis_name="c", subcore_axis_name="s")

    @functools.partial(
        pl.kernel,
        out_type=jax.ShapeDtypeStruct((_NC * npad,), jnp.float32),
        mesh=mesh,
        scratch_types=[
            pltpu.VMEM((nch, _K), jnp.int32),
            pltpu.VMEM((_K,), jnp.float32),
            pltpu.VMEM_SHARED((npad,), jnp.float32),
        ],
    )
    def deg_kernel(dst_hbm, z_hbm, ones_hbm, out_hbm, didx_v, ones_v, acc):
        c = lax.axis_index("c")
        s = lax.axis_index("s")
        wid = c * _NS + s
        r0 = s * rpt
        pltpu.sync_copy(z_hbm.at[pl.ds(r0, rpt)], acc.at[pl.ds(r0, rpt)])
        pltpu.sync_copy(ones_hbm, ones_v)
        pltpu.sync_copy(dst_hbm.at[pl.ds(wid * nch, nch)], didx_v)
        plsc.subcore_barrier()

        @pl.loop(0, nch)
        def _chunk(j):
            pltpu.sync_copy(ones_v, acc.at[didx_v.at[j]], add=True)

        plsc.subcore_barrier()
        pltpu.sync_copy(acc.at[pl.ds(r0, rpt)],
                        out_hbm.at[pl.ds(c * npad + r0, rpt)])

    return deg_kernel(dst2d, zeros1d, ones128)


def _scatter_parts(hs_ext, src2d, dst2d, npad, dout, nch0, nch1):
    nchmax = max(nch0, nch1)
    rpt = npad // _NS
    mesh = plsc.VectorSubcoreMesh(core_axis_name="c", subcore_axis_name="s")

    @functools.partial(
        pl.kernel,
        out_type=jax.ShapeDtypeStruct((_NC, npad, dout), jnp.float32),
        mesh=mesh,
        scratch_types=[
            pltpu.VMEM((nchmax, _K), jnp.int32),
            pltpu.VMEM((_K,), jnp.int32),
            pltpu.VMEM((_K,), jnp.int32),
            pltpu.VMEM((_K, dout), jnp.float32),
            pltpu.VMEM((_K, dout), jnp.float32),
            pltpu.VMEM_SHARED((npad, dout), jnp.float32),
            pltpu.SemaphoreType.DMA,
            pltpu.SemaphoreType.DMA,
            pltpu.SemaphoreType.DMA,
            pltpu.SemaphoreType.DMA,
        ],
    )
    def scat_kernel(hs_hbm, src_hbm, dst_hbm, out_hbm,
                    sidx_v, didx0, didx1, rows0, rows1, acc,
                    gsem0, gsem1, dsem0, dsem1):
        c = lax.axis_index("c")
        s = lax.axis_index("s")
        r0 = s * rpt
        row0 = jnp.where(c == 0, s * nch0, _NS * nch0 + s * nch1)
        ncl = jnp.where(c == 0, nch0, nch1)
        bufs = ((rows0, gsem0, didx0, dsem0), (rows1, gsem1, didx1, dsem1))
        pltpu.sync_copy(hs_hbm.at[pl.ds(r0, rpt)], acc.at[pl.ds(r0, rpt)])
        pltpu.sync_copy(src_hbm.at[pl.ds(row0, nchmax)], sidx_v)
        plsc.subcore_barrier()

        @pl.when(ncl >= 2)
        def _prologue():
            for b, (rows_b, gsem_b, didx_b, dsem_b) in enumerate(bufs):
                pltpu.async_copy(dst_hbm.at[row0 + b], didx_b, dsem_b)
                pltpu.async_copy(hs_hbm.at[sidx_v.at[b]], rows_b, gsem_b)

        @pl.loop(0, ncl - 2, step=2)
        def _chunk(j):
            for b, (rows_b, gsem_b, didx_b, dsem_b) in enumerate(bufs):
                cc = j + b
                pltpu.make_async_copy(dst_hbm.at[row0 + cc],
                                      didx_b, dsem_b).wait()
                pltpu.make_async_copy(hs_hbm.at[sidx_v.at[cc]],
                                      rows_b, gsem_b).wait()
                pltpu.sync_copy(rows_b, acc.at[didx_b], add=True)
                pltpu.async_copy(dst_hbm.at[row0 + cc + 2], didx_b, dsem_b)
                pltpu.async_copy(hs_hbm.at[sidx_v.at[cc + 2]], rows_b, gsem_b)

        @pl.when(ncl >= 2)
        def _epilogue():
            for b, (rows_b, gsem_b, didx_b, dsem_b) in enumerate(bufs):
                cc = ncl - 2 + b
                pltpu.make_async_copy(dst_hbm.at[row0 + cc],
                                      didx_b, dsem_b).wait()
                pltpu.make_async_copy(hs_hbm.at[sidx_v.at[cc]],
                                      rows_b, gsem_b).wait()
                pltpu.sync_copy(rows_b, acc.at[didx_b], add=True)

        plsc.subcore_barrier()
        pltpu.sync_copy(acc.at[pl.ds(r0, rpt)], out_hbm.at[c, pl.ds(r0, rpt)])

    return scat_kernel(hs_ext, src2d, dst2d)


def _to_col(m, blk):
    nr = blk // 128
    rows = lax.broadcasted_iota(jnp.int32, (blk, nr), 0) // 128
    sel = (rows == lax.broadcasted_iota(jnp.int32, (blk, nr), 1))
    c = jnp.dot(sel.astype(jnp.float32), m,
                preferred_element_type=jnp.float32)
    lane = lax.broadcasted_iota(jnp.int32, (blk, 128), 1)
    vmod = lax.broadcasted_iota(jnp.int32, (blk, 128), 0) % 128
    return jnp.sum(jnp.where(lane == vmod, c, 0.0), axis=1, keepdims=True)


def _dinv_block(d0_ref, d1_ref, blk):
    d = _to_col(d0_ref[...] + d1_ref[...] + 1.0, blk)
    return lax.rsqrt(jnp.maximum(d, 1.0))


def _linear(x_ext, W, deg0, deg1, npad, din, dout, blk):
    grid = (npad // blk,)
    dblk = blk // 128

    def body(x_ref, w_ref, d0_ref, d1_ref, o_ref):
        dinv = _dinv_block(d0_ref, d1_ref, blk)
        h = jnp.dot(x_ref[...], w_ref[...], preferred_element_type=jnp.float32)
        o_ref[...] = h * dinv

    return pl.pallas_call(
        body,
        grid=grid,
        in_specs=[
            pl.BlockSpec((blk, din), lambda i: (i, 0)),
            pl.BlockSpec((din, dout), lambda i: (0, 0)),
            pl.BlockSpec((dblk, 128), lambda i: (i, 0)),
            pl.BlockSpec((dblk, 128), lambda i: (i, 0)),
        ],
        out_specs=pl.BlockSpec((blk, dout), lambda i: (i, 0)),
        out_shape=jax.ShapeDtypeStruct((npad, dout), jnp.float32),
    )(x_ext, W, deg0, deg1)


def _agg_block(p0_ref, p1_ref, hs_ref, d0_ref, d1_ref, b_ref, blk):
    dinv = _dinv_block(d0_ref, d1_ref, blk)
    return (p0_ref[...] + p1_ref[...] - hs_ref[...]) * dinv + b_ref[...]


def _stats(p0, p1, hs_ext, deg0, deg1, b2, npad, n, dout, blk):
    grid = (npad // blk,)

    def body(p0_ref, p1_ref, hs_ref, d0_ref, d1_ref, b_ref, o_ref):
        i = pl.program_id(0)
        agg = _agg_block(p0_ref, p1_ref, hs_ref, d0_ref, d1_ref, b_ref, blk)
        rid = lax.broadcasted_iota(jnp.int32, (blk, 1), 0) + i * blk
        agg = jnp.where(rid < n, agg, 0.0)

        @pl.when(i == 0)
        def _():
            o_ref[...] = jnp.zeros_like(o_ref)

        o_ref[0:1, :] += jnp.sum(agg, axis=0, keepdims=True)
        o_ref[1:2, :] += jnp.sum(agg * agg, axis=0, keepdims=True)

    dblk = blk // 128
    return pl.pallas_call(
        body,
        grid=grid,
        in_specs=[
            pl.BlockSpec((blk, dout), lambda i: (i, 0)),
            pl.BlockSpec((blk, dout), lambda i: (i, 0)),
            pl.BlockSpec((blk, dout), lambda i: (i, 0)),
            pl.BlockSpec((dblk, 128), lambda i: (i, 0)),
            pl.BlockSpec((dblk, 128), lambda i: (i, 0)),
            pl.BlockSpec((1, dout), lambda i: (0, 0)),
        ],
        out_specs=pl.BlockSpec((8, dout), lambda i: (0, 0)),
        out_shape=jax.ShapeDtypeStruct((8, dout), jnp.float32),
    )(p0, p1, hs_ext, deg0, deg1, b2)


def _batchnorm(p0, p1, hs_ext, deg0, deg1, b2, g2, bt2, stats, npad, n, dout,
               blk):
    grid = (npad // blk,)
    inv_n = 1.0 / n

    def body(p0_ref, p1_ref, hs_ref, d0_ref, d1_ref, b_ref, g_ref, bt_ref,
             st_ref, o_ref):
        agg = _agg_block(p0_ref, p1_ref, hs_ref, d0_ref, d1_ref, b_ref, blk)
        mean = st_ref[0:1, :] * inv_n
        var = st_ref[1:2, :] * inv_n - mean * mean
        y = (agg - mean) * lax.rsqrt(var + _EPS)
        y = g_ref[...] * y + bt_ref[...]
        o_ref[...] = jnp.maximum(y, 0.0)

    dblk = blk // 128
    return pl.pallas_call(
        body,
        grid=grid,
        in_specs=[
            pl.BlockSpec((blk, dout), lambda i: (i, 0)),
            pl.BlockSpec((blk, dout), lambda i: (i, 0)),
            pl.BlockSpec((blk, dout), lambda i: (i, 0)),
            pl.BlockSpec((dblk, 128), lambda i: (i, 0)),
            pl.BlockSpec((dblk, 128), lambda i: (i, 0)),
            pl.BlockSpec((1, dout), lambda i: (0, 0)),
            pl.BlockSpec((1, dout), lambda i: (0, 0)),
            pl.BlockSpec((1, dout), lambda i: (0, 0)),
            pl.BlockSpec((8, dout), lambda i: (0, 0)),
        ],
        out_specs=pl.BlockSpec((blk, dout), lambda i: (i, 0)),
        out_shape=jax.ShapeDtypeStruct((npad, dout), jnp.float32),
    )(p0, p1, hs_ext, deg0, deg1, b2, g2, bt2, stats)


def kernel(x, edge_index, W, b, gamma, beta):
    n, din = x.shape
    dout = W.shape[1]
    e = edge_index.shape[1]

    blk = 1024
    npad = -(-(n + 1) // blk) * blk
    grp = _NW * _K * 2
    epad = -(-e // grp) * grp
    nrows = epad // _K
    tp = nrows // _NS
    nch0 = min(max(8 * int(round(_SPLIT0 * tp / 8)), 0), tp)
    nch1 = tp - nch0
    nchmax = max(nch0, nch1)

    pad = jnp.full((epad - e + nchmax * _K,), n, dtype=jnp.int32)
    src2d = jnp.concatenate([edge_index[0], pad]).reshape(-1, _K)
    dst2d = jnp.concatenate([edge_index[1], pad]).reshape(-1, _K)
    x_ext = jnp.zeros((npad, din), jnp.float32).at[:n].set(x)
    zeros1d = jnp.zeros((npad,), jnp.float32)
    ones128 = jnp.ones((_K,), jnp.float32)

    degflat = _deg_parts(dst2d, zeros1d, ones128, npad, nrows)
    deg0 = degflat[:npad].reshape(npad // 128, 128)
    deg1 = degflat[npad:].reshape(npad // 128, 128)

    hs_ext = _linear(x_ext, W, deg0, deg1, npad, din, dout, blk)

    parts = _scatter_parts(hs_ext, src2d, dst2d, npad, dout, nch0, nch1)
    p0, p1 = parts[0], parts[1]

    b2 = b.reshape(1, dout)
    g2 = gamma.reshape(1, dout)
    bt2 = beta.reshape(1, dout)
    stats = _stats(p0, p1, hs_ext, deg0, deg1, b2, npad, n, dout, blk)
    y = _batchnorm(p0, p1, hs_ext, deg0, deg1, b2, g2, bt2, stats, npad, n,
                   dout, blk)
    return y[:n]
